# Initial kernel scaffold; baseline (speedup 1.0000x reference)
import jax
import jax.numpy as jnp
from jax.experimental import pallas as pl


def kernel(x, edge_index, Wn, Ws, b, gamma, beta):
    raise NotImplementedError("write your pallas kernel here")


# speedup vs baseline: 4.3559x; 4.3559x over previous
import functools

import jax
import jax.numpy as jnp
from jax import lax
from jax.experimental import pallas as pl
from jax.experimental.pallas import tpu as pltpu
from jax.experimental.pallas import tpu_sc as plsc

N = 10000
E = 320000
C = 128
NB = 3

NC = 2
NS = 16
K = 128
EPW = E // (NC * NS)
STEPS = (EPW + K - 1) // K
EPW_PAD = STEPS * K
ROWS_PER_TILE = 632
NP = NS * ROWS_PER_TILE


def _edge_agg_body(y_hbm, src_hbm, dst_hbm, zeros_hbm, out_hbm,
                   src_v, dst_v, rows_v, acc, sem):
    c = lax.axis_index("c")
    s = lax.axis_index("s")
    r0 = s * ROWS_PER_TILE

    pltpu.sync_copy(zeros_hbm.at[pl.ds(r0, ROWS_PER_TILE)],
                    acc.at[pl.ds(r0, ROWS_PER_TILE)])
    pltpu.sync_copy(src_hbm.at[c, s], src_v)
    pltpu.sync_copy(dst_hbm.at[c, s], dst_v)
    plsc.subcore_barrier()

    def step(j, carry):
        pltpu.async_copy(y_hbm.at[src_v.at[j]], rows_v, sem).wait()
        pltpu.sync_copy(rows_v, acc.at[dst_v.at[j]], add=True)
        return carry

    lax.fori_loop(0, STEPS, step, 0)
    plsc.subcore_barrier()
    pltpu.sync_copy(acc.at[pl.ds(r0, ROWS_PER_TILE)],
                    out_hbm.at[c, pl.ds(r0, ROWS_PER_TILE)])


_edge_agg = pl.kernel(
    _edge_agg_body,
    out_type=jax.ShapeDtypeStruct((NC, NP, C), jnp.float32),
    mesh=plsc.VectorSubcoreMesh(core_axis_name="c", subcore_axis_name="s"),
    scratch_types=[
        pltpu.VMEM((STEPS, K), jnp.int32),
        pltpu.VMEM((STEPS, K), jnp.int32),
        pltpu.VMEM((K, C), jnp.float32),
        pltpu.VMEM_SHARED((NP, C), jnp.float32),
        pltpu.SemaphoreType.DMA,
    ],
)


def _mm2_body(h_ref, wn_ref, ws_ref, b_ref, y_ref, base_ref):
    h = h_ref[...]
    y_ref[...] = jnp.dot(h, wn_ref[...], preferred_element_type=jnp.float32)
    base_ref[...] = (jnp.dot(h, ws_ref[...], preferred_element_type=jnp.float32)
                     + b_ref[...])


_mm2 = pl.pallas_call(
    _mm2_body,
    out_shape=(jax.ShapeDtypeStruct((N, C), jnp.float32),
               jax.ShapeDtypeStruct((N, C), jnp.float32)),
)


def _bn_body(parts_ref, base_ref, g_ref, be_ref, idn_ref, o_ref, *, residual):
    t = parts_ref[0, :N, :] + parts_ref[1, :N, :] + base_ref[...]
    mu = jnp.mean(t, axis=0, keepdims=True)
    d = t - mu
    var = jnp.mean(d * d, axis=0, keepdims=True)
    out = d * lax.rsqrt(var + 1e-3) * g_ref[...] + be_ref[...]
    if residual:
        out = out + idn_ref[...]
    o_ref[...] = jnp.maximum(out, 0.0)


def _make_bn(residual):
    return pl.pallas_call(
        functools.partial(_bn_body, residual=residual),
        out_shape=jax.ShapeDtypeStruct((N, C), jnp.float32),
    )


_bn_plain = _make_bn(False)
_bn_res = _make_bn(True)


def kernel(x, edge_index, Wn, Ws, b, gamma, beta):
    src = edge_index[0].astype(jnp.int32)
    dst = edge_index[1].astype(jnp.int32)
    pad = EPW_PAD * NC * NS - E
    src = jnp.concatenate([src, jnp.zeros((pad,), jnp.int32)])
    dst = jnp.concatenate([dst, jnp.full((pad,), N, jnp.int32)])
    src_g = src.reshape(NC, NS, STEPS, K)
    dst_g = dst.reshape(NC, NS, STEPS, K)
    zeros = jnp.zeros((NP, C), jnp.float32)

    h = x
    for i in range(NB):
        identity = h
        for j in range(2):
            y, base = _mm2(h, Wn[i, j], Ws[i, j], b[i, j][None])
            parts = _edge_agg(y, src_g, dst_g, zeros)
            if j == 0:
                h = _bn_plain(parts, base, gamma[i, j][None], beta[i, j][None],
                              identity)
            else:
                h = _bn_res(parts, base, gamma[i, j][None], beta[i, j][None],
                            identity)
    return h
